# baseline (speedup 1.0000x reference)
import dataclasses
import functools

import numpy as np

import jax
import jax.numpy as jnp
from jax import lax
from jax.experimental import pallas as pl
from jax.experimental.pallas import tpu as pltpu
from jax.experimental.pallas import tpu_sc as plsc

NC = 2
NS = 16
NW = NC * NS
LANES = 16
CHUNK = 128


def _round_up(a, m):
    return (a + m - 1) // m * m


def _sc_compiler_params():
    cp = pltpu.CompilerParams()
    if "needs_layout_passes" in pltpu.CompilerParams.__dataclass_fields__:
        cp = dataclasses.replace(cp, needs_layout_passes=False)
    return cp


def _deg_kernel(dst2d, n_pad, ch0, ch1):
    mesh = plsc.VectorSubcoreMesh(core_axis_name="c", subcore_axis_name="s")
    ch_max = max(ch0, ch1)

    @functools.partial(
        pl.kernel,
        out_type=jax.ShapeDtypeStruct((NW, n_pad), jnp.float32),
        mesh=mesh,
        scratch_types=[
            pltpu.VMEM((ch_max, CHUNK), jnp.int32),
            pltpu.VMEM((n_pad,), jnp.float32),
        ],
        compiler_params=_sc_compiler_params(),
    )
    def k(dst_hbm, out_hbm, didx, degbuf):
        cid = lax.axis_index("c")
        sid = lax.axis_index("s")
        w = cid * NS + sid
        zeros = jnp.zeros((LANES,), jnp.float32)
        ones = jnp.ones((LANES,), jnp.float32)

        @pl.loop(0, n_pad, step=LANES)
        def _(i):
            degbuf[pl.ds(i, LANES)] = zeros

        def hist(base_chunk, nch):
            pltpu.sync_copy(
                dst_hbm.at[pl.ds(base_chunk, nch)], didx.at[pl.ds(0, nch)]
            )

            @pl.loop(0, nch)
            def _(c):
                for j in range(CHUNK // LANES):
                    idx = didx[c, pl.ds(j * LANES, LANES)]
                    plsc.addupdate_scatter(degbuf, [idx], ones)

        @pl.when(cid == 0)
        def _():
            hist(sid * ch0, ch0)

        @pl.when(cid == 1)
        def _():
            hist(NS * ch0 + sid * ch1, ch1)

        pltpu.sync_copy(degbuf, out_hbm.at[w])

    return k(dst2d)


def _matmul_kernel(x, w_mat, blk):
    n, d = x.shape

    def body(x_ref, w_ref, y_ref):
        y_ref[...] = lax.dot_general(
            x_ref[...], w_ref[...], (((1,), (1,)), ((), ())),
            preferred_element_type=jnp.float32,
            precision=lax.Precision.HIGHEST,
        )

    return pl.pallas_call(
        body,
        grid=(n // blk,),
        in_specs=[
            pl.BlockSpec((blk, d), lambda i: (i, 0)),
            pl.BlockSpec((d, d), lambda i: (0, 0)),
        ],
        out_specs=pl.BlockSpec((blk, d), lambda i: (i, 0)),
        out_shape=jax.ShapeDtypeStruct((n, d), jnp.float32),
    )(x, w_mat)


def _dis_kernel(deg_parts):
    n_pad = deg_parts.shape[1]
    blk = 1024
    assert n_pad % blk == 0

    def body(degp_ref, dis_ref):
        ones = jnp.ones((NW, 1), jnp.float32)
        deg = lax.dot_general(
            degp_ref[...], ones, (((0,), (0,)), ((), ())),
            preferred_element_type=jnp.float32,
            precision=lax.Precision.HIGHEST,
        )
        dis_ref[...] = lax.rsqrt(deg + 1.0)

    return pl.pallas_call(
        body,
        grid=(n_pad // blk,),
        in_specs=[pl.BlockSpec((NW, blk), lambda i: (0, i))],
        out_specs=pl.BlockSpec((blk, 1), lambda i: (i, 0)),
        out_shape=jax.ShapeDtypeStruct((n_pad, 1), jnp.float32),
    )(deg_parts)


def _prescale_kernel(dis, y, blk):
    n, d = y.shape

    def body(dis_ref, y_ref, yt_ref):
        yt_ref[...] = y_ref[...] * dis_ref[...]

    return pl.pallas_call(
        body,
        grid=(n // blk,),
        in_specs=[
            pl.BlockSpec((blk, 1), lambda i: (i, 0)),
            pl.BlockSpec((blk, d), lambda i: (i, 0)),
        ],
        out_specs=pl.BlockSpec((blk, d), lambda i: (i, 0)),
        out_shape=jax.ShapeDtypeStruct((n, d), jnp.float32),
    )(dis, y)


def _propagate_kernel(xt, src2d, dst2d, n, n_pad, ch0, ch1, group):
    d = xt.shape[1]
    zero_copies = n_pad // NS // CHUNK
    out_rows = (n // NS) // 8 * 8
    out_rows_last = n - (NS - 1) * out_rows
    mesh = plsc.VectorSubcoreMesh(core_axis_name="c", subcore_axis_name="s")

    @functools.partial(
        pl.kernel,
        out_type=jax.ShapeDtypeStruct((NC, n, d), jnp.float32),
        mesh=mesh,
        scratch_types=[
            pltpu.VMEM((group, CHUNK), jnp.int32),
            pltpu.VMEM((group, CHUNK), jnp.int32),
            pltpu.VMEM((CHUNK, d), jnp.float32),
            pltpu.VMEM((CHUNK, d), jnp.float32),
            pltpu.VMEM_SHARED((n_pad, d), jnp.float32),
            pltpu.SemaphoreType.DMA,
            pltpu.SemaphoreType.DMA,
        ],
        compiler_params=_sc_compiler_params(),
    )
    def k(xt_hbm, src_hbm, dst_hbm, out_hbm, sidx, didx, rows_a, rows_b,
          h_sh, sem_a, sem_b):
        cid = lax.axis_index("c")
        sid = lax.axis_index("s")
        zeros = jnp.zeros((LANES,), jnp.float32)
        bufs = ((rows_a, sem_a), (rows_b, sem_b))

        @pl.loop(0, CHUNK)
        def _(r):
            for j in range(d // LANES):
                rows_a[r, pl.ds(j * LANES, LANES)] = zeros

        @pl.loop(0, zero_copies)
        def _(z):
            pltpu.sync_copy(
                rows_a, h_sh.at[pl.ds((sid * zero_copies + z) * CHUNK, CHUNK)]
            )

        plsc.subcore_barrier()

        def edge_pipe(base_chunk, nch):
            @pl.loop(0, nch, step=group)
            def _(g):
                pltpu.sync_copy(src_hbm.at[pl.ds(base_chunk + g, group)], sidx)
                pltpu.sync_copy(dst_hbm.at[pl.ds(base_chunk + g, group)], didx)
                pltpu.async_copy(xt_hbm.at[sidx.at[0]], rows_a, sem_a)
                pltpu.async_copy(xt_hbm.at[sidx.at[1]], rows_b, sem_b)

                @pl.loop(0, group, step=2)
                def _(c):
                    for i, (rows, sem) in enumerate(bufs):
                        pltpu.make_async_copy(
                            xt_hbm.at[pl.ds(0, CHUNK)], rows, sem
                        ).wait()
                        pltpu.sync_copy(rows, h_sh.at[didx.at[c + i]], add=True)

                        @pl.when(c + i + 2 < group)
                        def _():
                            pltpu.async_copy(
                                xt_hbm.at[sidx.at[c + i + 2]], rows, sem
                            )

        @pl.when(cid == 0)
        def _():
            edge_pipe(sid * ch0, ch0)

        @pl.when(cid == 1)
        def _():
            edge_pipe(NS * ch0 + sid * ch1, ch1)

        plsc.subcore_barrier()

        @pl.when(sid < NS - 1)
        def _():
            pltpu.sync_copy(
                h_sh.at[pl.ds(sid * out_rows, out_rows)],
                out_hbm.at[cid, pl.ds(sid * out_rows, out_rows)],
            )

        @pl.when(sid == NS - 1)
        def _():
            pltpu.sync_copy(
                h_sh.at[pl.ds((NS - 1) * out_rows, out_rows_last)],
                out_hbm.at[cid, pl.ds((NS - 1) * out_rows, out_rows_last)],
            )

    return k(xt, src2d, dst2d)


def _final_kernel(dis, yt, s, b_row, blk):
    n, d = yt.shape

    def body(dis_ref, yt_ref, s_ref, b_ref, out_ref):
        h = (s_ref[0] + s_ref[1] + yt_ref[...]) * dis_ref[...]
        z = h + b_ref[...]
        out_ref[...] = jnp.where(z >= 0.0, z, 0.1 * z)

    return pl.pallas_call(
        body,
        grid=(n // blk,),
        in_specs=[
            pl.BlockSpec((blk, 1), lambda i: (i, 0)),
            pl.BlockSpec((blk, d), lambda i: (i, 0)),
            pl.BlockSpec((NC, blk, d), lambda i: (0, i, 0)),
            pl.BlockSpec((1, d), lambda i: (0, 0)),
        ],
        out_specs=pl.BlockSpec((blk, d), lambda i: (i, 0)),
        out_shape=jax.ShapeDtypeStruct((n, d), jnp.float32),
    )(dis, yt, s, b_row)


def kernel(x, edge_index, W, b):
    n, d = x.shape
    e = edge_index.shape[1]
    e_pad = _round_up(e, NW * CHUNK * 8)
    ch_pair = e_pad // CHUNK // NS
    group = ch_pair // 4
    ch0 = 2 * group
    ch1 = ch_pair - ch0
    n_pad = _round_up(n + 1, NS * CHUNK)

    pad = e_pad - e
    assert e % CHUNK == 0 and n_pad - n >= 128
    pad_i = np.arange(pad, dtype=np.int32)
    src_pad = np.asarray(pad_i % n, np.int32).reshape(pad // CHUNK, CHUNK)
    dst_pad = np.asarray(n + (pad_i & 127), np.int32).reshape(pad // CHUNK, CHUNK)
    src2d = jnp.concatenate([edge_index[0].reshape(e // CHUNK, CHUNK),
                             jnp.asarray(src_pad)])
    dst2d = jnp.concatenate([edge_index[1].reshape(e // CHUNK, CHUNK),
                             jnp.asarray(dst_pad)])

    blk = 2000
    y = _matmul_kernel(x, W, blk)
    deg_parts = _deg_kernel(dst2d, n_pad, ch0, ch1)
    dis = _dis_kernel(deg_parts)
    yt = _prescale_kernel(dis, y, blk)
    s = _propagate_kernel(yt, src2d, dst2d, n, n_pad, ch0, ch1, group)
    return _final_kernel(dis, yt, s, b.reshape(1, d), blk)


# speedup vs baseline: 1.0678x; 1.0678x over previous
import dataclasses
import functools

import numpy as np

import jax
import jax.numpy as jnp
from jax import lax
from jax.experimental import pallas as pl
from jax.experimental.pallas import tpu as pltpu
from jax.experimental.pallas import tpu_sc as plsc

NC = 2
NS = 16
NW = NC * NS
LANES = 16
CHUNK = 128


def _round_up(a, m):
    return (a + m - 1) // m * m


def _sc_compiler_params():
    cp = pltpu.CompilerParams()
    if "needs_layout_passes" in pltpu.CompilerParams.__dataclass_fields__:
        cp = dataclasses.replace(cp, needs_layout_passes=False)
    return cp


def _deg_kernel(e3d, n_pad, ch0, ch1):
    mesh = plsc.VectorSubcoreMesh(core_axis_name="c", subcore_axis_name="s")
    ch_max = max(ch0, ch1)

    @functools.partial(
        pl.kernel,
        out_type=jax.ShapeDtypeStruct((NW, n_pad), jnp.float32),
        mesh=mesh,
        scratch_types=[
            pltpu.VMEM((ch_max, CHUNK), jnp.int32),
            pltpu.VMEM((n_pad,), jnp.float32),
        ],
        compiler_params=_sc_compiler_params(),
    )
    def k(edges_hbm, out_hbm, didx, degbuf):
        cid = lax.axis_index("c")
        sid = lax.axis_index("s")
        w = cid * NS + sid
        zeros = jnp.zeros((LANES,), jnp.float32)
        ones = jnp.ones((LANES,), jnp.float32)

        @pl.loop(0, n_pad, step=LANES)
        def _(i):
            degbuf[pl.ds(i, LANES)] = zeros

        def hist(base_chunk, nch):
            pltpu.sync_copy(
                edges_hbm.at[1, pl.ds(base_chunk, nch)], didx.at[pl.ds(0, nch)]
            )

            @pl.loop(0, nch)
            def _(c):
                for j in range(CHUNK // LANES):
                    idx = didx[c, pl.ds(j * LANES, LANES)]
                    plsc.addupdate_scatter(degbuf, [idx], ones)

        @pl.when(cid == 0)
        def _():
            hist(sid * ch0, ch0)

        @pl.when(cid == 1)
        def _():
            hist(NS * ch0 + sid * ch1, ch1)

        pltpu.sync_copy(degbuf, out_hbm.at[w])

    return k(e3d)


def _matmul_kernel(x, w_mat, blk):
    n, d = x.shape

    def body(x_ref, w_ref, y_ref):
        y_ref[...] = lax.dot_general(
            x_ref[...], w_ref[...], (((1,), (1,)), ((), ())),
            preferred_element_type=jnp.float32,
            precision=lax.Precision.HIGHEST,
        )

    return pl.pallas_call(
        body,
        grid=(n // blk,),
        in_specs=[
            pl.BlockSpec((blk, d), lambda i: (i, 0)),
            pl.BlockSpec((d, d), lambda i: (0, 0)),
        ],
        out_specs=pl.BlockSpec((blk, d), lambda i: (i, 0)),
        out_shape=jax.ShapeDtypeStruct((n, d), jnp.float32),
    )(x, w_mat)


def _dis_kernel(deg_parts):
    n_pad = deg_parts.shape[1]

    def body(degp_ref, dis_ref):
        ones = jnp.ones((NW, 1), jnp.float32)
        deg = lax.dot_general(
            degp_ref[...], ones, (((0,), (0,)), ((), ())),
            preferred_element_type=jnp.float32,
            precision=lax.Precision.HIGHEST,
        )
        dis_ref[...] = lax.rsqrt(deg + 1.0)

    return pl.pallas_call(
        body,
        out_shape=jax.ShapeDtypeStruct((n_pad, 1), jnp.float32),
    )(deg_parts)


def _prescale_kernel(dis, y, blk):
    n, d = y.shape

    def body(dis_ref, y_ref, yt_ref):
        yt_ref[...] = y_ref[...] * dis_ref[...]

    return pl.pallas_call(
        body,
        grid=(n // blk,),
        in_specs=[
            pl.BlockSpec((blk, 1), lambda i: (i, 0)),
            pl.BlockSpec((blk, d), lambda i: (i, 0)),
        ],
        out_specs=pl.BlockSpec((blk, d), lambda i: (i, 0)),
        out_shape=jax.ShapeDtypeStruct((n, d), jnp.float32),
    )(dis, y)


def _propagate_kernel(xt, e3d, n, n_pad, ch0, ch1, group):
    d = xt.shape[1]
    zero_copies = n_pad // NS // CHUNK
    out_rows = (n // NS) // 8 * 8
    out_rows_last = n - (NS - 1) * out_rows
    mesh = plsc.VectorSubcoreMesh(core_axis_name="c", subcore_axis_name="s")

    @functools.partial(
        pl.kernel,
        out_type=jax.ShapeDtypeStruct((NC, n, d), jnp.float32),
        mesh=mesh,
        scratch_types=[
            pltpu.VMEM((group, CHUNK), jnp.int32),
            pltpu.VMEM((group, CHUNK), jnp.int32),
            pltpu.VMEM((CHUNK, d), jnp.float32),
            pltpu.VMEM((CHUNK, d), jnp.float32),
            pltpu.VMEM_SHARED((n_pad, d), jnp.float32),
            pltpu.SemaphoreType.DMA,
            pltpu.SemaphoreType.DMA,
        ],
        compiler_params=_sc_compiler_params(),
    )
    def k(xt_hbm, edges_hbm, out_hbm, sidx, didx, rows_a, rows_b,
          h_sh, sem_a, sem_b):
        cid = lax.axis_index("c")
        sid = lax.axis_index("s")
        zeros = jnp.zeros((LANES,), jnp.float32)
        bufs = ((rows_a, sem_a), (rows_b, sem_b))

        @pl.loop(0, CHUNK)
        def _(r):
            for j in range(d // LANES):
                rows_a[r, pl.ds(j * LANES, LANES)] = zeros

        @pl.loop(0, zero_copies)
        def _(z):
            pltpu.sync_copy(
                rows_a, h_sh.at[pl.ds((sid * zero_copies + z) * CHUNK, CHUNK)]
            )

        plsc.subcore_barrier()

        def edge_pipe(base_chunk, nch):
            @pl.loop(0, nch, step=group)
            def _(g):
                pltpu.sync_copy(edges_hbm.at[0, pl.ds(base_chunk + g, group)],
                                sidx)
                pltpu.sync_copy(edges_hbm.at[1, pl.ds(base_chunk + g, group)],
                                didx)
                pltpu.async_copy(xt_hbm.at[sidx.at[0]], rows_a, sem_a)
                pltpu.async_copy(xt_hbm.at[sidx.at[1]], rows_b, sem_b)

                @pl.loop(0, group, step=2)
                def _(c):
                    for i, (rows, sem) in enumerate(bufs):
                        pltpu.make_async_copy(
                            xt_hbm.at[pl.ds(0, CHUNK)], rows, sem
                        ).wait()
                        pltpu.sync_copy(rows, h_sh.at[didx.at[c + i]], add=True)

                        @pl.when(c + i + 2 < group)
                        def _():
                            pltpu.async_copy(
                                xt_hbm.at[sidx.at[c + i + 2]], rows, sem
                            )

        @pl.when(cid == 0)
        def _():
            edge_pipe(sid * ch0, ch0)

        @pl.when(cid == 1)
        def _():
            edge_pipe(NS * ch0 + sid * ch1, ch1)

        plsc.subcore_barrier()

        @pl.when(sid < NS - 1)
        def _():
            pltpu.sync_copy(
                h_sh.at[pl.ds(sid * out_rows, out_rows)],
                out_hbm.at[cid, pl.ds(sid * out_rows, out_rows)],
            )

        @pl.when(sid == NS - 1)
        def _():
            pltpu.sync_copy(
                h_sh.at[pl.ds((NS - 1) * out_rows, out_rows_last)],
                out_hbm.at[cid, pl.ds((NS - 1) * out_rows, out_rows_last)],
            )

    return k(xt, e3d)


def _final_kernel(dis, yt, s, b_row, blk):
    n, d = yt.shape

    def body(dis_ref, yt_ref, s_ref, b_ref, out_ref):
        h = (s_ref[0] + s_ref[1] + yt_ref[...]) * dis_ref[...]
        z = h + b_ref[...]
        out_ref[...] = jnp.where(z >= 0.0, z, 0.1 * z)

    return pl.pallas_call(
        body,
        grid=(n // blk,),
        in_specs=[
            pl.BlockSpec((blk, 1), lambda i: (i, 0)),
            pl.BlockSpec((blk, d), lambda i: (i, 0)),
            pl.BlockSpec((NC, blk, d), lambda i: (0, i, 0)),
            pl.BlockSpec((1, d), lambda i: (0, 0)),
        ],
        out_specs=pl.BlockSpec((blk, d), lambda i: (i, 0)),
        out_shape=jax.ShapeDtypeStruct((n, d), jnp.float32),
    )(dis, yt, s, b_row)


def kernel(x, edge_index, W, b):
    n, d = x.shape
    e = edge_index.shape[1]
    e_pad = _round_up(e, NW * CHUNK * 8)
    ch_pair = e_pad // CHUNK // NS
    group = ch_pair // 4
    ch0 = 2 * group
    ch1 = ch_pair - ch0
    n_pad = _round_up(n + 1, NS * CHUNK)

    pad = e_pad - e
    assert e % CHUNK == 0 and n_pad - n >= 128
    pad_i = np.arange(pad, dtype=np.int32)
    pad3d = np.stack([
        np.asarray(pad_i % n, np.int32).reshape(pad // CHUNK, CHUNK),
        np.asarray(n + (pad_i & 127), np.int32).reshape(pad // CHUNK, CHUNK),
    ])
    e3d = jnp.concatenate(
        [edge_index.reshape(2, e // CHUNK, CHUNK), jnp.asarray(pad3d)], axis=1
    )

    blk = 2000
    y = _matmul_kernel(x, W, blk)
    deg_parts = _deg_kernel(e3d, n_pad, ch0, ch1)
    dis = _dis_kernel(deg_parts)
    yt = _prescale_kernel(dis, y, blk)
    s = _propagate_kernel(yt, e3d, n, n_pad, ch0, ch1, group)
    return _final_kernel(dis, yt, s, b.reshape(1, d), blk)


# speedup vs baseline: 1.1016x; 1.0316x over previous
import dataclasses
import functools

import numpy as np

import jax
import jax.numpy as jnp
from jax import lax
from jax.experimental import pallas as pl
from jax.experimental.pallas import tpu as pltpu
from jax.experimental.pallas import tpu_sc as plsc

NC = 2
NS = 16
NW = NC * NS
LANES = 16
CHUNK = 64
NBUF = 4


def _round_up(a, m):
    return (a + m - 1) // m * m


def _sc_compiler_params():
    cp = pltpu.CompilerParams()
    if "needs_layout_passes" in pltpu.CompilerParams.__dataclass_fields__:
        cp = dataclasses.replace(cp, needs_layout_passes=False)
    return cp


def _deg_kernel(e3d, n_pad, ch0, ch1):
    mesh = plsc.VectorSubcoreMesh(core_axis_name="c", subcore_axis_name="s")
    ch_max = max(ch0, ch1)

    @functools.partial(
        pl.kernel,
        out_type=jax.ShapeDtypeStruct((NW, n_pad), jnp.float32),
        mesh=mesh,
        scratch_types=[
            pltpu.VMEM((ch_max, CHUNK), jnp.int32),
            pltpu.VMEM((n_pad,), jnp.float32),
        ],
        compiler_params=_sc_compiler_params(),
    )
    def k(edges_hbm, out_hbm, didx, degbuf):
        cid = lax.axis_index("c")
        sid = lax.axis_index("s")
        w = cid * NS + sid
        zeros = jnp.zeros((LANES,), jnp.float32)
        ones = jnp.ones((LANES,), jnp.float32)

        @pl.loop(0, n_pad, step=LANES)
        def _(i):
            degbuf[pl.ds(i, LANES)] = zeros

        def hist(base_chunk, nch):
            pltpu.sync_copy(
                edges_hbm.at[1, pl.ds(base_chunk, nch)], didx.at[pl.ds(0, nch)]
            )

            @pl.loop(0, nch)
            def _(c):
                for j in range(CHUNK // LANES):
                    idx = didx[c, pl.ds(j * LANES, LANES)]
                    plsc.addupdate_scatter(degbuf, [idx], ones)

        @pl.when(cid == 0)
        def _():
            hist(sid * ch0, ch0)

        @pl.when(cid == 1)
        def _():
            hist(NS * ch0 + sid * ch1, ch1)

        pltpu.sync_copy(degbuf, out_hbm.at[w])

    return k(e3d)


def _matmul_kernel(x, w_mat, blk):
    n, d = x.shape

    def body(x_ref, w_ref, y_ref):
        y_ref[...] = lax.dot_general(
            x_ref[...], w_ref[...], (((1,), (1,)), ((), ())),
            preferred_element_type=jnp.float32,
            precision=lax.Precision.HIGHEST,
        )

    return pl.pallas_call(
        body,
        grid=(n // blk,),
        in_specs=[
            pl.BlockSpec((blk, d), lambda i: (i, 0)),
            pl.BlockSpec((d, d), lambda i: (0, 0)),
        ],
        out_specs=pl.BlockSpec((blk, d), lambda i: (i, 0)),
        out_shape=jax.ShapeDtypeStruct((n, d), jnp.float32),
    )(x, w_mat)


def _dis_kernel(deg_parts):
    n_pad = deg_parts.shape[1]

    def body(degp_ref, dis_ref):
        ones = jnp.ones((NW, 1), jnp.float32)
        deg = lax.dot_general(
            degp_ref[...], ones, (((0,), (0,)), ((), ())),
            preferred_element_type=jnp.float32,
            precision=lax.Precision.HIGHEST,
        )
        dis_ref[...] = lax.rsqrt(deg + 1.0)

    return pl.pallas_call(
        body,
        out_shape=jax.ShapeDtypeStruct((n_pad, 1), jnp.float32),
    )(deg_parts)


def _prescale_kernel(dis, y, blk):
    n, d = y.shape

    def body(dis_ref, y_ref, yt_ref):
        yt_ref[...] = y_ref[...] * dis_ref[...]

    return pl.pallas_call(
        body,
        grid=(n // blk,),
        in_specs=[
            pl.BlockSpec((blk, 1), lambda i: (i, 0)),
            pl.BlockSpec((blk, d), lambda i: (i, 0)),
        ],
        out_specs=pl.BlockSpec((blk, d), lambda i: (i, 0)),
        out_shape=jax.ShapeDtypeStruct((n, d), jnp.float32),
    )(dis, y)


def _propagate_kernel(xt, e3d, n, n_pad, ch0, ch1, group):
    d = xt.shape[1]
    zero_copies = n_pad // NS // CHUNK
    out_rows = (n // NS) // 8 * 8
    out_rows_last = n - (NS - 1) * out_rows
    mesh = plsc.VectorSubcoreMesh(core_axis_name="c", subcore_axis_name="s")

    @functools.partial(
        pl.kernel,
        out_type=jax.ShapeDtypeStruct((NC, n, d), jnp.float32),
        mesh=mesh,
        scratch_types=[
            pltpu.VMEM((group, CHUNK), jnp.int32),
            pltpu.VMEM((group, CHUNK), jnp.int32),
        ] + [pltpu.VMEM((CHUNK, d), jnp.float32)] * NBUF
          + [pltpu.VMEM_SHARED((n_pad, d), jnp.float32)]
          + [pltpu.SemaphoreType.DMA] * NBUF,
        compiler_params=_sc_compiler_params(),
    )
    def k(xt_hbm, edges_hbm, out_hbm, sidx, didx, *rest):
        rows_bufs = rest[:NBUF]
        h_sh = rest[NBUF]
        sems = rest[NBUF + 1:]
        rows_a = rows_bufs[0]
        cid = lax.axis_index("c")
        sid = lax.axis_index("s")
        zeros = jnp.zeros((LANES,), jnp.float32)
        bufs = tuple(zip(rows_bufs, sems))

        @pl.loop(0, CHUNK)
        def _(r):
            for j in range(d // LANES):
                rows_a[r, pl.ds(j * LANES, LANES)] = zeros

        @pl.loop(0, zero_copies)
        def _(z):
            pltpu.sync_copy(
                rows_a, h_sh.at[pl.ds((sid * zero_copies + z) * CHUNK, CHUNK)]
            )

        plsc.subcore_barrier()

        def edge_pipe(base_chunk, nch):
            @pl.loop(0, nch, step=group)
            def _(g):
                pltpu.sync_copy(edges_hbm.at[0, pl.ds(base_chunk + g, group)],
                                sidx)
                pltpu.sync_copy(edges_hbm.at[1, pl.ds(base_chunk + g, group)],
                                didx)
                for i, (rows, sem) in enumerate(bufs):
                    pltpu.async_copy(xt_hbm.at[sidx.at[i]], rows, sem)

                @pl.loop(0, group, step=NBUF)
                def _(c):
                    for i, (rows, sem) in enumerate(bufs):
                        pltpu.make_async_copy(
                            xt_hbm.at[pl.ds(0, CHUNK)], rows, sem
                        ).wait()
                        pltpu.sync_copy(rows, h_sh.at[didx.at[c + i]], add=True)

                        @pl.when(c + i + NBUF < group)
                        def _():
                            pltpu.async_copy(
                                xt_hbm.at[sidx.at[c + i + NBUF]], rows, sem
                            )

        @pl.when(cid == 0)
        def _():
            edge_pipe(sid * ch0, ch0)

        @pl.when(cid == 1)
        def _():
            edge_pipe(NS * ch0 + sid * ch1, ch1)

        plsc.subcore_barrier()

        @pl.when(sid < NS - 1)
        def _():
            pltpu.sync_copy(
                h_sh.at[pl.ds(sid * out_rows, out_rows)],
                out_hbm.at[cid, pl.ds(sid * out_rows, out_rows)],
            )

        @pl.when(sid == NS - 1)
        def _():
            pltpu.sync_copy(
                h_sh.at[pl.ds((NS - 1) * out_rows, out_rows_last)],
                out_hbm.at[cid, pl.ds((NS - 1) * out_rows, out_rows_last)],
            )

    return k(xt, e3d)


def _final_kernel(dis, yt, s, b_row, blk):
    n, d = yt.shape

    def body(dis_ref, yt_ref, s_ref, b_ref, out_ref):
        h = (s_ref[0] + s_ref[1] + yt_ref[...]) * dis_ref[...]
        z = h + b_ref[...]
        out_ref[...] = jnp.where(z >= 0.0, z, 0.1 * z)

    return pl.pallas_call(
        body,
        grid=(n // blk,),
        in_specs=[
            pl.BlockSpec((blk, 1), lambda i: (i, 0)),
            pl.BlockSpec((blk, d), lambda i: (i, 0)),
            pl.BlockSpec((NC, blk, d), lambda i: (0, i, 0)),
            pl.BlockSpec((1, d), lambda i: (0, 0)),
        ],
        out_specs=pl.BlockSpec((blk, d), lambda i: (i, 0)),
        out_shape=jax.ShapeDtypeStruct((n, d), jnp.float32),
    )(dis, yt, s, b_row)


def kernel(x, edge_index, W, b):
    n, d = x.shape
    e = edge_index.shape[1]
    e_pad = _round_up(e, NW * CHUNK * 8)
    ch_pair = e_pad // CHUNK // NS
    group = ch_pair // 8
    ch0 = 4 * group
    ch1 = ch_pair - ch0
    n_pad = _round_up(n + 1, NS * CHUNK)

    pad = e_pad - e
    assert e % CHUNK == 0 and n_pad - n >= 128
    pad_i = np.arange(pad, dtype=np.int32)
    pad3d = np.stack([
        np.asarray(pad_i % n, np.int32).reshape(pad // CHUNK, CHUNK),
        np.asarray(n + (pad_i & 127), np.int32).reshape(pad // CHUNK, CHUNK),
    ])
    e3d = jnp.concatenate(
        [edge_index.reshape(2, e // CHUNK, CHUNK), jnp.asarray(pad3d)], axis=1
    )

    blk = 2000
    y = _matmul_kernel(x, W, blk)
    deg_parts = _deg_kernel(e3d, n_pad, ch0, ch1)
    dis = _dis_kernel(deg_parts)
    yt = _prescale_kernel(dis, y, blk)
    s = _propagate_kernel(yt, e3d, n, n_pad, ch0, ch1, group)
    return _final_kernel(dis, yt, s, b.reshape(1, d), blk)
